# baseline (speedup 1.0000x reference)
import functools

import jax
import jax.numpy as jnp
from jax import lax
from jax.experimental import pallas as pl
from jax.experimental.pallas import tpu as pltpu
from jax.experimental.pallas import tpu_sc as plsc

N = 10000
E = 320000
D = 128
DH = D // 2
G = 32
T = 4
H_OUT = 64
CUTOFF = 8.0

NC = 2
NS = 16
NW = NC * NS
CH = 80
EPW = E // NW
EPT = E // NS
NCH = EPT // CH
NPAD = 10240
RPW = NPAD // NW
ECH = 80
NECH = RPW // ECH

_mesh = plsc.VectorSubcoreMesh(
    core_axis_name="c", subcore_axis_name="s", num_cores=NC, num_subcores=NS)


@functools.partial(
    pl.kernel,
    out_type=(
        jax.ShapeDtypeStruct((NPAD, D), jnp.float32),
        jax.ShapeDtypeStruct((E,), jnp.float32),
    ),
    mesh=_mesh,
    scratch_types=[
        pltpu.VMEM((RPW,), jnp.int32),
        pltpu.VMEM((ECH, D), jnp.float32),
        pltpu.VMEM((N,), jnp.float32),
        pltpu.VMEM((N,), jnp.float32),
        pltpu.VMEM((N,), jnp.float32),
        pltpu.VMEM((EPW,), jnp.int32),
        pltpu.VMEM((EPW,), jnp.int32),
        pltpu.VMEM((EPW,), jnp.float32),
        pltpu.SemaphoreType.DMA,
    ],
    compiler_params=pltpu.CompilerParams(needs_layout_passes=False),
)
def _sc_pre(z_hbm, xc_hbm, yc_hbm, zc_hbm, src_hbm, dst_hbm, embed_hbm,
            x0_hbm, s_hbm,
            zv, rows, xv, yv, zzv, siv, div, sv, sem):
    w = lax.axis_index("s") * NC + lax.axis_index("c")
    pltpu.sync_copy(z_hbm.at[pl.ds(w * RPW, RPW)], zv)
    for c in range(NECH):
        pltpu.async_copy(embed_hbm.at[zv.at[pl.ds(c * ECH, ECH)]], rows, sem).wait()
        pltpu.sync_copy(rows, x0_hbm.at[pl.ds(w * RPW + c * ECH, ECH)])
    pltpu.sync_copy(xc_hbm, xv)
    pltpu.sync_copy(yc_hbm, yv)
    pltpu.sync_copy(zc_hbm, zzv)
    base = w * EPW
    pltpu.sync_copy(src_hbm.at[pl.ds(base, EPW)], siv)
    pltpu.sync_copy(dst_hbm.at[pl.ds(base, EPW)], div)

    def body(g, _):
        sl = pl.ds(g * 16, 16)
        si = siv[sl]
        di = div[sl]
        dx = plsc.load_gather(xv, [di]) - plsc.load_gather(xv, [si])
        dy = plsc.load_gather(yv, [di]) - plsc.load_gather(yv, [si])
        dz = plsc.load_gather(zzv, [di]) - plsc.load_gather(zzv, [si])
        sv[sl] = dx * dx + dy * dy + dz * dz
        return 0

    lax.fori_loop(0, EPW // 16, body, 0)
    pltpu.sync_copy(sv, s_hbm.at[pl.ds(base, EPW)])


_BE = 2560

_DNT = (((0,), (0,)), ((), ()))


def _filter_body(srow_ref, wfalo_ref, wfahi_ref, wr1_ref, wr2lo_ref,
                 wr2hi_ref, br1c_ref, out_ref):
    width = CUTOFF / (G - 1)
    rrow = jnp.sqrt(srow_ref[...] + 1e-8)
    offs = lax.broadcasted_iota(jnp.int32, (G, _BE), 0).astype(jnp.float32)
    diff = (rrow - offs * width) * (1.0 / width)
    _L2E = 1.4426950408889634
    fijT = jnp.exp2((-0.5 * _L2E) * diff * diff)
    cutrow = 0.5 * (jnp.cos(rrow * (jnp.pi / CUTOFF)) + 1.0)
    cutrow = cutrow * (rrow < CUTOFF).astype(jnp.float32)
    preT = lax.dot_general(wr1_ref[...], fijT, _DNT,
                           preferred_element_type=jnp.float32) + br1c_ref[...]
    ex = jnp.exp2(jnp.minimum(preT * _L2E, 120.0))
    actT = (preT * cutrow) * (ex / (1.0 + ex))
    fija = jnp.concatenate([fijT * cutrow, cutrow], axis=0)
    wlo = (lax.dot_general(fija, wfalo_ref[...], _DNT,
                           preferred_element_type=jnp.float32)
           + lax.dot_general(actT, wr2lo_ref[...], _DNT,
                             preferred_element_type=jnp.float32))
    whi = (lax.dot_general(fija, wfahi_ref[...], _DNT,
                           preferred_element_type=jnp.float32)
           + lax.dot_general(actT, wr2hi_ref[...], _DNT,
                             preferred_element_type=jnp.float32))
    out_ref[...] = jnp.concatenate([wlo, whi], axis=1)


def _make_filter_call():
    full = lambda i: (0, 0)
    row = lambda i: (i, 0)
    return pl.pallas_call(
        _filter_body,
        grid=(E // _BE,),
        in_specs=[
            pl.BlockSpec((1, _BE), lambda i: (0, i)),
            pl.BlockSpec((G + 1, DH), full),
            pl.BlockSpec((G + 1, DH), full),
            pl.BlockSpec((G, D), full),
            pl.BlockSpec((D, DH), full),
            pl.BlockSpec((D, DH), full),
            pl.BlockSpec((D, 1), full),
        ],
        out_specs=pl.BlockSpec((_BE, D), row),
        out_shape=jax.ShapeDtypeStruct((E, D), jnp.float32),
    )


_BN = 1000


def _h_body(x_ref, te_ref, win_ref, bin_ref, h_ref):
    h_ref[...] = (jnp.dot(x_ref[...] + te_ref[...], win_ref[...],
                          preferred_element_type=jnp.float32) + bin_ref[...])


def _make_h_call():
    full = lambda i: (0, 0)
    row = lambda i: (i, 0)
    return pl.pallas_call(
        _h_body,
        grid=(N // _BN,),
        in_specs=[
            pl.BlockSpec((_BN, D), row),
            pl.BlockSpec((1, D), full),
            pl.BlockSpec((D, D), full),
            pl.BlockSpec((1, D), full),
        ],
        out_specs=pl.BlockSpec((_BN, D), row),
        out_shape=jax.ShapeDtypeStruct((N, D), jnp.float32),
    )


def _upd_body(x_ref, alo_ref, ahi_ref, wolo_ref, wohi_ref, bout_ref, te_ref,
              win_ref, bin_ref, xn_ref, h_ref):
    xn = (x_ref[...]
          + jnp.dot(alo_ref[...], wolo_ref[...], preferred_element_type=jnp.float32)
          + jnp.dot(ahi_ref[...], wohi_ref[...], preferred_element_type=jnp.float32)
          + bout_ref[...])
    xn_ref[...] = xn
    h_ref[...] = (jnp.dot(xn + te_ref[...], win_ref[...],
                          preferred_element_type=jnp.float32) + bin_ref[...])


def _make_upd_call():
    full = lambda i: (0, 0)
    row = lambda i: (i, 0)
    return pl.pallas_call(
        _upd_body,
        grid=(N // _BN,),
        in_specs=[
            pl.BlockSpec((_BN, D), row),
            pl.BlockSpec((_BN, DH), row),
            pl.BlockSpec((_BN, DH), row),
            pl.BlockSpec((DH, D), full),
            pl.BlockSpec((DH, D), full),
            pl.BlockSpec((1, D), full),
            pl.BlockSpec((1, D), full),
            pl.BlockSpec((D, D), full),
            pl.BlockSpec((1, D), full),
        ],
        out_specs=(pl.BlockSpec((_BN, D), row), pl.BlockSpec((_BN, D), row)),
        out_shape=(jax.ShapeDtypeStruct((N, D), jnp.float32),
                   jax.ShapeDtypeStruct((N, D), jnp.float32)),
    )


def _readout_body(x_ref, wo1_ref, bo1_ref, wo2_ref, out_ref):
    i = pl.program_id(0)
    t1 = jnp.dot(x_ref[...], wo1_ref[...],
                 preferred_element_type=jnp.float32) + bo1_ref[...]
    ex = jnp.exp2(jnp.minimum(t1 * 1.4426950408889634, 120.0))
    t1 = t1 * (ex / (1.0 + ex))
    ps = jnp.sum(t1 * wo2_ref[...], axis=(0, 1), keepdims=True)

    @pl.when(i == 0)
    def _():
        out_ref[...] = ps

    @pl.when(i > 0)
    def _():
        out_ref[...] += ps


def _make_readout_call():
    full = lambda i: (0, 0)
    return pl.pallas_call(
        _readout_body,
        grid=(N // _BN,),
        in_specs=[
            pl.BlockSpec((_BN, D), lambda i: (i, 0)),
            pl.BlockSpec((D, H_OUT), full),
            pl.BlockSpec((1, H_OUT), full),
            pl.BlockSpec((1, H_OUT), full),
        ],
        out_specs=pl.BlockSpec((1, 1), full),
        out_shape=jax.ShapeDtypeStruct((1, 1), jnp.float32),
    )


@functools.partial(
    pl.kernel,
    out_type=(
        jax.ShapeDtypeStruct((NPAD, DH), jnp.float32),
        jax.ShapeDtypeStruct((NPAD, DH), jnp.float32),
    ),
    mesh=_mesh,
    scratch_types=[
        pltpu.VMEM((NCH, CH), jnp.int32),
        pltpu.VMEM((NCH, CH), jnp.int32),
        pltpu.VMEM((CH, DH), jnp.float32),
        pltpu.VMEM((CH, DH), jnp.float32),
        pltpu.VMEM((CH, DH), jnp.float32),
        pltpu.VMEM((CH, DH), jnp.float32),
        pltpu.VMEM((128, DH), jnp.float32),
        pltpu.VMEM_SHARED((NPAD, DH), jnp.float32),
        pltpu.SemaphoreType.DMA,
        pltpu.SemaphoreType.DMA,
        pltpu.SemaphoreType.DMA,
        pltpu.SemaphoreType.DMA,
        pltpu.SemaphoreType.DMA,
        pltpu.SemaphoreType.DMA,
    ],
    compiler_params=pltpu.CompilerParams(needs_layout_passes=False,
                                         use_tc_tiling_on_sc=False),
)
def _sc_step(h2_hbm, w_hbm, src3_hbm, dst3_hbm,
             alo_hbm, ahi_hbm,
             siv, div, ra, rb, wa, wb, zb, agg, sga, sgb, swa, swb, ssa, ssb):
    cid = lax.axis_index("c")
    sid = lax.axis_index("s")
    ebase = sid * EPT
    csl = pl.ds(cid * DH, DH)

    pltpu.sync_copy(src3_hbm.at[sid], siv)
    pltpu.sync_copy(dst3_hbm.at[sid], div)

    def sxf(c, _):
        for g in range(CH // 16):
            sl = pl.ds(g * 16, 16)
            siv[c, sl] = siv[c, sl] * 2 + cid
        return 0

    lax.fori_loop(0, NCH, sxf, 0)

    def start(c, rbuf, gsem, wbuf, wsem):
        pltpu.async_copy(h2_hbm.at[siv.at[c]], rbuf, gsem)
        pltpu.async_copy(w_hbm.at[pl.ds(ebase + c * CH, CH), csl], wbuf, wsem)

    def wait(c, rbuf, gsem, wbuf, wsem):
        pltpu.make_async_copy(h2_hbm.at[siv.at[c]], rbuf, gsem).wait()
        pltpu.make_async_copy(w_hbm.at[pl.ds(ebase + c * CH, CH), csl], wbuf,
                              wsem).wait()

    def mulc(rbuf, wbuf):
        def _mul(e, _):
            for k in range(DH // 16):
                sl = pl.ds(k * 16, 16)
                rbuf[e, sl] = rbuf[e, sl] * wbuf[e, sl]
            return 0

        lax.fori_loop(0, CH, _mul, 0)

    def scat_start(c, rbuf, ssem):
        pltpu.async_copy(rbuf, agg.at[div.at[c]], ssem, add=True)

    def scat_wait(c, rbuf, ssem):
        pltpu.make_async_copy(rbuf, agg.at[div.at[c]], ssem).wait()

    start(0, ra, sga, wa, swa)

    def zrow(i, _):
        for k in range(DH // 16):
            zb[i, pl.ds(k * 16, 16)] = jnp.zeros((16,), jnp.float32)
        return 0

    lax.fori_loop(0, 128, zrow, 0)
    for q in range(5):
        pltpu.sync_copy(zb, agg.at[pl.ds(sid * 640 + q * 128, 128)])
    plsc.subcore_barrier()

    def pair(jj, _):
        c0 = jj * 2
        c1 = c0 + 1

        @pl.when(jj > 0)
        def _():
            scat_wait(c1 - 2, rb, ssb)

        start(c1, rb, sgb, wb, swb)
        wait(c0, ra, sga, wa, swa)
        mulc(ra, wa)
        scat_start(c0, ra, ssa)
        wait(c1, rb, sgb, wb, swb)
        mulc(rb, wb)
        scat_wait(c0, ra, ssa)

        @pl.when(c0 + 2 < NCH)
        def _():
            start(c0 + 2, ra, sga, wa, swa)

        scat_start(c1, rb, ssb)
        return 0

    lax.fori_loop(0, NCH // 2, pair, 0)
    scat_wait(NCH - 1, rb, ssb)

    plsc.subcore_barrier()
    sl640 = pl.ds(sid * 640, 640)

    @pl.when(cid == 0)
    def _():
        pltpu.sync_copy(agg.at[sl640], alo_hbm.at[sl640])

    @pl.when(cid == 1)
    def _():
        pltpu.sync_copy(agg.at[sl640], ahi_hbm.at[sl640])


def kernel(z, xyz, edge_index, embed, Wf, bf, Wr1, br1, Wr2, br2,
           Win, b_in, Wout, b_out, time_emb, Wo1, bo1, Wo2, bo2):
    src = edge_index[0]
    dst = edge_index[1]
    z_pad = jnp.concatenate([z.astype(jnp.int32),
                             jnp.zeros((NPAD - N,), jnp.int32)])
    xc = xyz[:, 0]
    yc = xyz[:, 1]
    zc = xyz[:, 2]

    x0_pad, s = _sc_pre(z_pad, xc, yc, zc, src, dst, embed)
    x = x0_pad[:N]

    r1 = lambda a: a.reshape(1, -1)
    bfr2 = bf + br2
    wfa = jnp.concatenate([Wf, bfr2.reshape(1, D)], axis=0)
    W = _make_filter_call()(
        s.reshape(1, E), wfa[:, :DH], wfa[:, DH:], Wr1,
        Wr2[:, :DH], Wr2[:, DH:], br1.reshape(D, 1))

    src3 = src.reshape(NS, NCH, CH)
    dst3 = dst.reshape(NS, NCH, CH)

    h_call = _make_h_call()
    upd_call = _make_upd_call()

    h = h_call(x, r1(time_emb[0]), Win, r1(b_in))
    for t in range(T):
        alo, ahi = _sc_step(h.reshape(2 * N, DH), W, src3, dst3)
        te_next = time_emb[t + 1] if t + 1 < T else jnp.zeros((D,), jnp.float32)
        x, h = upd_call(x, alo, ahi, Wout[:DH], Wout[DH:],
                        r1(b_out), r1(te_next), Win, r1(b_in))

    out = _make_readout_call()(x, Wo1, r1(bo1), Wo2.T)
    energy = out[0, 0] + jnp.float32(N) * bo2[0]
    return energy


# speedup vs baseline: 1.1236x; 1.1236x over previous
import functools

import jax
import jax.numpy as jnp
from jax import lax
from jax.experimental import pallas as pl
from jax.experimental.pallas import tpu as pltpu
from jax.experimental.pallas import tpu_sc as plsc

N = 10000
E = 320000
D = 128
DH = D // 2
G = 32
T = 4
H_OUT = 64
CUTOFF = 8.0

NC = 2
NS = 16
NW = NC * NS
CH = 80
EPW = E // NW
EPT = E // NS
NCH = EPT // CH
NPAD = 10240
RPW = NPAD // NW
ECH = 80
NECH = RPW // ECH

_mesh = plsc.VectorSubcoreMesh(
    core_axis_name="c", subcore_axis_name="s", num_cores=NC, num_subcores=NS)


@functools.partial(
    pl.kernel,
    out_type=(
        jax.ShapeDtypeStruct((NPAD, D), jnp.float32),
        jax.ShapeDtypeStruct((E,), jnp.float32),
    ),
    mesh=_mesh,
    scratch_types=[
        pltpu.VMEM((RPW,), jnp.int32),
        pltpu.VMEM((ECH, D), jnp.float32),
        pltpu.VMEM((N,), jnp.float32),
        pltpu.VMEM((N,), jnp.float32),
        pltpu.VMEM((N,), jnp.float32),
        pltpu.VMEM((EPW,), jnp.int32),
        pltpu.VMEM((EPW,), jnp.int32),
        pltpu.VMEM((EPW,), jnp.float32),
        pltpu.SemaphoreType.DMA,
    ],
    compiler_params=pltpu.CompilerParams(needs_layout_passes=False),
)
def _sc_pre(z_hbm, xc_hbm, yc_hbm, zc_hbm, src_hbm, dst_hbm, embed_hbm,
            x0_hbm, s_hbm,
            zv, rows, xv, yv, zzv, siv, div, sv, sem):
    w = lax.axis_index("s") * NC + lax.axis_index("c")
    pltpu.sync_copy(z_hbm.at[pl.ds(w * RPW, RPW)], zv)
    for c in range(NECH):
        pltpu.async_copy(embed_hbm.at[zv.at[pl.ds(c * ECH, ECH)]], rows, sem).wait()
        pltpu.sync_copy(rows, x0_hbm.at[pl.ds(w * RPW + c * ECH, ECH)])
    pltpu.sync_copy(xc_hbm, xv)
    pltpu.sync_copy(yc_hbm, yv)
    pltpu.sync_copy(zc_hbm, zzv)
    base = w * EPW
    pltpu.sync_copy(src_hbm.at[pl.ds(base, EPW)], siv)
    pltpu.sync_copy(dst_hbm.at[pl.ds(base, EPW)], div)

    def body(g, _):
        sl = pl.ds(g * 16, 16)
        si = siv[sl]
        di = div[sl]
        dx = plsc.load_gather(xv, [di]) - plsc.load_gather(xv, [si])
        dy = plsc.load_gather(yv, [di]) - plsc.load_gather(yv, [si])
        dz = plsc.load_gather(zzv, [di]) - plsc.load_gather(zzv, [si])
        sv[sl] = dx * dx + dy * dy + dz * dz
        return 0

    lax.fori_loop(0, EPW // 16, body, 0)
    pltpu.sync_copy(sv, s_hbm.at[pl.ds(base, EPW)])


_BE = 2560

_DNT = (((0,), (0,)), ((), ()))


def _filter_body(srow_ref, wfalo_ref, wfahi_ref, wr1_ref, wr2lo_ref,
                 wr2hi_ref, br1c_ref, out_ref):
    width = CUTOFF / (G - 1)
    rrow = jnp.sqrt(srow_ref[...] + 1e-8)
    offs = lax.broadcasted_iota(jnp.int32, (G, _BE), 0).astype(jnp.float32)
    diff = (rrow - offs * width) * (1.0 / width)
    _L2E = 1.4426950408889634
    fijT = jnp.exp2((-0.5 * _L2E) * diff * diff)
    cutrow = 0.5 * (jnp.cos(rrow * (jnp.pi / CUTOFF)) + 1.0)
    cutrow = cutrow * (rrow < CUTOFF).astype(jnp.float32)
    preT = lax.dot_general(wr1_ref[...], fijT, _DNT,
                           preferred_element_type=jnp.float32) + br1c_ref[...]
    ex = jnp.exp2(jnp.minimum(preT * _L2E, 120.0))
    actT = (preT * cutrow) * (ex / (1.0 + ex))
    fija = jnp.concatenate([fijT * cutrow, cutrow], axis=0)
    wlo = (lax.dot_general(fija, wfalo_ref[...], _DNT,
                           preferred_element_type=jnp.float32)
           + lax.dot_general(actT, wr2lo_ref[...], _DNT,
                             preferred_element_type=jnp.float32))
    whi = (lax.dot_general(fija, wfahi_ref[...], _DNT,
                           preferred_element_type=jnp.float32)
           + lax.dot_general(actT, wr2hi_ref[...], _DNT,
                             preferred_element_type=jnp.float32))
    out_ref[...] = jnp.concatenate([wlo, whi], axis=1)


def _make_filter_call():
    full = lambda i: (0, 0)
    row = lambda i: (i, 0)
    return pl.pallas_call(
        _filter_body,
        grid=(E // _BE,),
        in_specs=[
            pl.BlockSpec((1, _BE), lambda i: (0, i)),
            pl.BlockSpec((G + 1, DH), full),
            pl.BlockSpec((G + 1, DH), full),
            pl.BlockSpec((G, D), full),
            pl.BlockSpec((D, DH), full),
            pl.BlockSpec((D, DH), full),
            pl.BlockSpec((D, 1), full),
        ],
        out_specs=pl.BlockSpec((_BE, D), row),
        out_shape=jax.ShapeDtypeStruct((E, D), jnp.float32),
    )


_BN = 1000


def _h_body(x_ref, te_ref, win_ref, bin_ref, h_ref):
    h_ref[...] = (jnp.dot(x_ref[...] + te_ref[...], win_ref[...],
                          preferred_element_type=jnp.float32) + bin_ref[...])


def _make_h_call():
    full = lambda i: (0, 0)
    row = lambda i: (i, 0)
    return pl.pallas_call(
        _h_body,
        grid=(N // _BN,),
        in_specs=[
            pl.BlockSpec((_BN, D), row),
            pl.BlockSpec((1, D), full),
            pl.BlockSpec((D, D), full),
            pl.BlockSpec((1, D), full),
        ],
        out_specs=pl.BlockSpec((_BN, D), row),
        out_shape=jax.ShapeDtypeStruct((N, D), jnp.float32),
    )


def _upd_body(x_ref, alo_ref, ahi_ref, wolo_ref, wohi_ref, bout_ref, te_ref,
              win_ref, bin_ref, xn_ref, h_ref):
    xn = (x_ref[...]
          + jnp.dot(alo_ref[...], wolo_ref[...], preferred_element_type=jnp.float32)
          + jnp.dot(ahi_ref[...], wohi_ref[...], preferred_element_type=jnp.float32)
          + bout_ref[...])
    xn_ref[...] = xn
    h_ref[...] = (jnp.dot(xn + te_ref[...], win_ref[...],
                          preferred_element_type=jnp.float32) + bin_ref[...])


def _make_upd_call():
    full = lambda i: (0, 0)
    row = lambda i: (i, 0)
    return pl.pallas_call(
        _upd_body,
        grid=(N // _BN,),
        in_specs=[
            pl.BlockSpec((_BN, D), row),
            pl.BlockSpec((_BN, DH), row),
            pl.BlockSpec((_BN, DH), row),
            pl.BlockSpec((DH, D), full),
            pl.BlockSpec((DH, D), full),
            pl.BlockSpec((1, D), full),
            pl.BlockSpec((1, D), full),
            pl.BlockSpec((D, D), full),
            pl.BlockSpec((1, D), full),
        ],
        out_specs=(pl.BlockSpec((_BN, D), row), pl.BlockSpec((_BN, D), row)),
        out_shape=(jax.ShapeDtypeStruct((N, D), jnp.float32),
                   jax.ShapeDtypeStruct((N, D), jnp.float32)),
    )


def _readout_body(x_ref, wo1_ref, bo1_ref, wo2_ref, out_ref):
    i = pl.program_id(0)
    t1 = jnp.dot(x_ref[...], wo1_ref[...],
                 preferred_element_type=jnp.float32) + bo1_ref[...]
    ex = jnp.exp2(jnp.minimum(t1 * 1.4426950408889634, 120.0))
    t1 = t1 * (ex / (1.0 + ex))
    ps = jnp.sum(t1 * wo2_ref[...], axis=(0, 1), keepdims=True)

    @pl.when(i == 0)
    def _():
        out_ref[...] = ps

    @pl.when(i > 0)
    def _():
        out_ref[...] += ps


def _make_readout_call():
    full = lambda i: (0, 0)
    return pl.pallas_call(
        _readout_body,
        grid=(N // _BN,),
        in_specs=[
            pl.BlockSpec((_BN, D), lambda i: (i, 0)),
            pl.BlockSpec((D, H_OUT), full),
            pl.BlockSpec((1, H_OUT), full),
            pl.BlockSpec((1, H_OUT), full),
        ],
        out_specs=pl.BlockSpec((1, 1), full),
        out_shape=jax.ShapeDtypeStruct((1, 1), jnp.float32),
    )


@functools.partial(
    pl.kernel,
    out_type=(
        jax.ShapeDtypeStruct((NPAD, DH), jnp.float32),
        jax.ShapeDtypeStruct((NPAD, DH), jnp.float32),
    ),
    mesh=_mesh,
    scratch_types=[
        pltpu.VMEM((NCH, CH), jnp.int32),
        pltpu.VMEM((NCH, CH), jnp.int32),
        pltpu.VMEM((CH, DH), jnp.float32),
        pltpu.VMEM((CH, DH), jnp.float32),
        pltpu.VMEM((CH, DH), jnp.float32),
        pltpu.VMEM((CH, DH), jnp.float32),
        pltpu.VMEM((128, DH), jnp.float32),
        pltpu.VMEM_SHARED((NPAD, DH), jnp.float32),
        pltpu.SemaphoreType.DMA,
        pltpu.SemaphoreType.DMA,
        pltpu.SemaphoreType.DMA,
        pltpu.SemaphoreType.DMA,
        pltpu.SemaphoreType.DMA,
        pltpu.SemaphoreType.DMA,
    ],
    compiler_params=pltpu.CompilerParams(needs_layout_passes=False,
                                         use_tc_tiling_on_sc=False),
)
def _sc_step(h2_hbm, w_hbm, src3_hbm, dst3_hbm,
             alo_hbm, ahi_hbm,
             siv, div, ra, rb, wa, wb, zb, agg, sga, sgb, swa, swb, ssa, ssb):
    cid = lax.axis_index("c")
    sid = lax.axis_index("s")
    ebase = sid * EPT
    csl = pl.ds(cid * DH, DH)

    pltpu.sync_copy(src3_hbm.at[sid], siv)
    pltpu.sync_copy(dst3_hbm.at[sid], div)

    def sxf(c, _):
        for g in range(CH // 16):
            sl = pl.ds(g * 16, 16)
            siv[c, sl] = siv[c, sl] * 2 + cid
        return 0

    lax.fori_loop(0, NCH, sxf, 0)

    def start(c, rbuf, gsem, wbuf, wsem):
        pltpu.async_copy(h2_hbm.at[siv.at[c]], rbuf, gsem)
        pltpu.async_copy(w_hbm.at[pl.ds(ebase + c * CH, CH), csl], wbuf, wsem)

    def wait(c, rbuf, gsem, wbuf, wsem):
        pltpu.make_async_copy(h2_hbm.at[siv.at[c]], rbuf, gsem).wait()
        pltpu.make_async_copy(w_hbm.at[pl.ds(ebase + c * CH, CH), csl], wbuf,
                              wsem).wait()

    def mulc(rbuf, wbuf):
        def _mul(e, _):
            for k in range(DH // 16):
                sl = pl.ds(k * 16, 16)
                rbuf[e, sl] = rbuf[e, sl] * wbuf[e, sl]
            return 0

        lax.fori_loop(0, CH, _mul, 0)

    def scat_start(c, rbuf, ssem):
        pltpu.async_copy(rbuf, agg.at[div.at[c]], ssem, add=True)

    def scat_wait(c, rbuf, ssem):
        pltpu.make_async_copy(rbuf, agg.at[div.at[c]], ssem).wait()

    start(0, ra, sga, wa, swa)

    def zrow(i, _):
        for k in range(DH // 16):
            zb[i, pl.ds(k * 16, 16)] = jnp.zeros((16,), jnp.float32)
        return 0

    lax.fori_loop(0, 128, zrow, 0)
    for q in range(5):
        pltpu.sync_copy(zb, agg.at[pl.ds(sid * 640 + q * 128, 128)])
    plsc.subcore_barrier()

    def pair(jj, _):
        c0 = jj * 2
        c1 = c0 + 1
        start(c1, rb, sgb, wb, swb)
        wait(c0, ra, sga, wa, swa)
        mulc(ra, wa)
        pltpu.sync_copy(ra, agg.at[div.at[c0]], add=True)

        @pl.when(c0 + 2 < NCH)
        def _():
            start(c0 + 2, ra, sga, wa, swa)

        wait(c1, rb, sgb, wb, swb)
        mulc(rb, wb)
        pltpu.sync_copy(rb, agg.at[div.at[c1]], add=True)
        return 0

    lax.fori_loop(0, NCH // 2, pair, 0)

    plsc.subcore_barrier()
    sl640 = pl.ds(sid * 640, 640)

    @pl.when(cid == 0)
    def _():
        pltpu.sync_copy(agg.at[sl640], alo_hbm.at[sl640])

    @pl.when(cid == 1)
    def _():
        pltpu.sync_copy(agg.at[sl640], ahi_hbm.at[sl640])


def kernel(z, xyz, edge_index, embed, Wf, bf, Wr1, br1, Wr2, br2,
           Win, b_in, Wout, b_out, time_emb, Wo1, bo1, Wo2, bo2):
    src = edge_index[0]
    dst = edge_index[1]
    z_pad = jnp.concatenate([z.astype(jnp.int32),
                             jnp.zeros((NPAD - N,), jnp.int32)])
    xc = xyz[:, 0]
    yc = xyz[:, 1]
    zc = xyz[:, 2]

    x0_pad, s = _sc_pre(z_pad, xc, yc, zc, src, dst, embed)
    x = x0_pad[:N]

    r1 = lambda a: a.reshape(1, -1)
    bfr2 = bf + br2
    wfa = jnp.concatenate([Wf, bfr2.reshape(1, D)], axis=0)
    W = _make_filter_call()(
        s.reshape(1, E), wfa[:, :DH], wfa[:, DH:], Wr1,
        Wr2[:, :DH], Wr2[:, DH:], br1.reshape(D, 1))

    src3 = src.reshape(NS, NCH, CH)
    dst3 = dst.reshape(NS, NCH, CH)

    h_call = _make_h_call()
    upd_call = _make_upd_call()

    h = h_call(x, r1(time_emb[0]), Win, r1(b_in))
    for t in range(T):
        alo, ahi = _sc_step(h.reshape(2 * N, DH), W, src3, dst3)
        te_next = time_emb[t + 1] if t + 1 < T else jnp.zeros((D,), jnp.float32)
        x, h = upd_call(x, alo, ahi, Wout[:DH], Wout[DH:],
                        r1(b_out), r1(te_next), Win, r1(b_in))

    out = _make_readout_call()(x, Wo1, r1(bo1), Wo2.T)
    energy = out[0, 0] + jnp.float32(N) * bo2[0]
    return energy
